# Initial kernel scaffold; baseline (speedup 1.0000x reference)
import jax
import jax.numpy as jnp
from jax.experimental import pallas as pl


def kernel(x, edge_index, W1, b1, W2, b2):
    raise NotImplementedError("write your pallas kernel here")


# speedup vs baseline: 13.5637x; 13.5637x over previous
import functools

import jax
import jax.numpy as jnp
from jax import lax
from jax.experimental import pallas as pl
from jax.experimental.pallas import tpu as pltpu
from jax.experimental.pallas import tpu_sc as plsc

NC = 2
NS = 16
NW = NC * NS
K = 80
L = 16


def _make_deg_kernel(n_nodes, n_edges):
    ept = n_edges // NW

    mesh = plsc.VectorSubcoreMesh(core_axis_name="c", subcore_axis_name="s")

    @functools.partial(
        pl.kernel,
        out_type=jax.ShapeDtypeStruct((NW * n_nodes,), jnp.float32),
        mesh=mesh,
        scratch_types=[
            pltpu.VMEM((ept,), jnp.int32),
            pltpu.VMEM((n_nodes,), jnp.float32),
        ],
        compiler_params=pltpu.CompilerParams(needs_layout_passes=False),
    )
    def deg_kernel(dst_hbm, zeros_hbm, out_hbm, dstv, degl):
        c = lax.axis_index("c")
        s = lax.axis_index("s")
        w = c * NS + s
        pltpu.sync_copy(zeros_hbm, degl)
        pltpu.sync_copy(dst_hbm.at[pl.ds(w * ept, ept)], dstv)
        ones = jnp.full((L,), 1.0, jnp.float32)

        def body(i, carry):
            idx = dstv[pl.ds(i * L, L)]
            plsc.addupdate_scatter(degl, [idx], ones)
            return carry

        lax.fori_loop(0, ept // L, body, 0)
        pltpu.sync_copy(degl, out_hbm.at[pl.ds(w * n_nodes, n_nodes)])

    return deg_kernel


def _make_agg_kernel(n_nodes, n_edges, width):
    ept = n_edges // NW
    chunks = ept // K
    rows_pt = (n_nodes // NS) // 8 * 8
    tail = n_nodes - NS * rows_pt

    mesh = plsc.VectorSubcoreMesh(core_axis_name="c", subcore_axis_name="s")

    @functools.partial(
        pl.kernel,
        out_type=jax.ShapeDtypeStruct((NC, n_nodes, width), jnp.float32),
        mesh=mesh,
        scratch_types=[
            pltpu.VMEM((K,), jnp.int32),
            pltpu.VMEM((K,), jnp.int32),
            pltpu.VMEM((K, width), jnp.float32),
            pltpu.VMEM_SHARED((n_nodes, width), jnp.float32),
            pltpu.SemaphoreType.DMA,
        ],
    )
    def agg_kernel(vals_hbm, src_hbm, dst_hbm, zeros_hbm, out_hbm,
                   srcv, dstv, rowsv, acc, sem):
        c = lax.axis_index("c")
        s = lax.axis_index("s")
        ebase = (c * NS + s) * ept
        rbase = s * rows_pt
        pltpu.sync_copy(zeros_hbm.at[pl.ds(0, rows_pt)],
                        acc.at[pl.ds(rbase, rows_pt)])

        @pl.when(s == 0)
        def _():
            pltpu.sync_copy(zeros_hbm.at[pl.ds(0, tail)],
                            acc.at[pl.ds(NS * rows_pt, tail)])

        plsc.subcore_barrier()

        def body(j, carry):
            off = ebase + j * K
            pltpu.sync_copy(src_hbm.at[pl.ds(off, K)], srcv)
            pltpu.sync_copy(dst_hbm.at[pl.ds(off, K)], dstv)
            pltpu.async_copy(vals_hbm.at[srcv], rowsv, sem).wait()
            pltpu.sync_copy(rowsv, acc.at[dstv], add=True)
            return carry

        lax.fori_loop(0, chunks, body, 0)
        plsc.subcore_barrier()
        pltpu.sync_copy(acc.at[pl.ds(rbase, rows_pt)],
                        out_hbm.at[c, pl.ds(rbase, rows_pt)])

        @pl.when(s == 0)
        def _():
            pltpu.sync_copy(acc.at[pl.ds(NS * rows_pt, tail)],
                            out_hbm.at[c, pl.ds(NS * rows_pt, tail)])

    return agg_kernel


def _dinv(degp_blk):
    deg = 1.0 + jnp.sum(degp_blk, axis=1)
    return lax.rsqrt(deg)[:, None]


def _tc1_body(x_ref, w1_ref, dp_ref, hs_ref):
    d = _dinv(dp_ref[...])
    h = jnp.dot(x_ref[...], w1_ref[...], preferred_element_type=jnp.float32)
    hs_ref[...] = h * d


def _tc2_body(a0_ref, a1_ref, hs_ref, dp_ref, b1_ref, out_ref):
    d = _dinv(dp_ref[...])
    z = jnp.maximum(d * (a0_ref[...] + a1_ref[...] + hs_ref[...]) + b1_ref[...],
                    0.0)
    out_ref[...] = z * d


def _tc3_body(a0_ref, a1_ref, zs_ref, dp_ref, w2_ref, b2_ref, out_ref):
    d = _dinv(dp_ref[...])
    g = d * (a0_ref[...] + a1_ref[...] + zs_ref[...])
    out_ref[...] = (jnp.dot(g, w2_ref[...], preferred_element_type=jnp.float32)
                    + b2_ref[...])


def kernel(x, edge_index, W1, b1, W2, b2):
    n, dx = x.shape
    e = edge_index.shape[1]
    h = W1.shape[1]
    c_out = W2.shape[1]
    src = edge_index[0]
    dst = edge_index[1]

    zeros_n = jnp.zeros((n,), jnp.float32)
    zeros_h = jnp.zeros((n // NS + 8, h), jnp.float32)
    b1r = b1.reshape(1, h)
    b2r = b2.reshape(1, c_out)

    bm = 1000
    grid = (n // bm,)
    row_blk = lambda w: pl.BlockSpec((bm, w), lambda i: (i, 0))
    dp_blk = pl.BlockSpec((bm, NW), lambda i: (i, 0))
    full_blk = lambda r, w: pl.BlockSpec((r, w), lambda i: (0, 0))

    deg_p = _make_deg_kernel(n, e)(dst, zeros_n).reshape(NW, n).T

    hs1 = pl.pallas_call(
        _tc1_body,
        grid=grid,
        in_specs=[row_blk(dx), full_blk(dx, h), dp_blk],
        out_specs=row_blk(h),
        out_shape=jax.ShapeDtypeStruct((n, h), jnp.float32),
    )(x, W1, deg_p)

    agg = _make_agg_kernel(n, e, h)
    acc1 = agg(hs1, src, dst, zeros_h)

    z1s = pl.pallas_call(
        _tc2_body,
        grid=grid,
        in_specs=[row_blk(h), row_blk(h), row_blk(h), dp_blk, full_blk(1, h)],
        out_specs=row_blk(h),
        out_shape=jax.ShapeDtypeStruct((n, h), jnp.float32),
    )(acc1[0], acc1[1], hs1, deg_p, b1r)

    acc2 = agg(z1s, src, dst, zeros_h)

    out = pl.pallas_call(
        _tc3_body,
        grid=grid,
        in_specs=[row_blk(h), row_blk(h), row_blk(h), dp_blk,
                  full_blk(h, c_out), full_blk(1, c_out)],
        out_specs=row_blk(c_out),
        out_shape=jax.ShapeDtypeStruct((n, c_out), jnp.float32),
    )(acc2[0], acc2[1], z1s, deg_p, W2, b2r)

    return out


# speedup vs baseline: 20.8448x; 1.5368x over previous
import functools

import jax
import jax.numpy as jnp
from jax import lax
from jax.experimental import pallas as pl
from jax.experimental.pallas import tpu as pltpu
from jax.experimental.pallas import tpu_sc as plsc

NC = 2
NS = 16
NW = NC * NS
K = 80
L = 16


def _make_deg_kernel(n_nodes, n_edges):
    ept = n_edges // NW

    mesh = plsc.VectorSubcoreMesh(core_axis_name="c", subcore_axis_name="s")

    @functools.partial(
        pl.kernel,
        out_type=jax.ShapeDtypeStruct((NW * n_nodes,), jnp.float32),
        mesh=mesh,
        scratch_types=[
            pltpu.VMEM((ept,), jnp.int32),
            pltpu.VMEM((n_nodes,), jnp.float32),
        ],
        compiler_params=pltpu.CompilerParams(needs_layout_passes=False),
    )
    def deg_kernel(dst_hbm, zeros_hbm, out_hbm, dstv, degl):
        c = lax.axis_index("c")
        s = lax.axis_index("s")
        w = c * NS + s
        pltpu.sync_copy(zeros_hbm, degl)
        pltpu.sync_copy(dst_hbm.at[pl.ds(w * ept, ept)], dstv)
        ones = jnp.full((L,), 1.0, jnp.float32)

        def body(i, carry):
            idx = dstv[pl.ds(i * L, L)]
            plsc.addupdate_scatter(degl, [idx], ones)
            return carry

        lax.fori_loop(0, ept // L, body, 0)
        pltpu.sync_copy(degl, out_hbm.at[pl.ds(w * n_nodes, n_nodes)])

    return deg_kernel


def _make_agg_kernel(n_nodes, n_edges, width):
    ept = n_edges // NW
    chunks = ept // K
    rows_pt = (n_nodes // NS) // 8 * 8
    tail = n_nodes - NS * rows_pt

    mesh = plsc.VectorSubcoreMesh(core_axis_name="c", subcore_axis_name="s")

    @functools.partial(
        pl.kernel,
        out_type=jax.ShapeDtypeStruct((NC, n_nodes, width), jnp.float32),
        mesh=mesh,
        scratch_types=[
            pltpu.VMEM((2, K), jnp.int32),
            pltpu.VMEM((2, K), jnp.int32),
            pltpu.VMEM((2, K, width), jnp.float32),
            pltpu.VMEM_SHARED((n_nodes, width), jnp.float32),
            pltpu.SemaphoreType.DMA((2,)),
        ],
    )
    def agg_kernel(vals_hbm, src_hbm, dst_hbm, zeros_hbm, out_hbm,
                   srcv, dstv, rowsv, acc, sems):
        c = lax.axis_index("c")
        s = lax.axis_index("s")
        ebase = (c * NS + s) * ept
        rbase = s * rows_pt
        pltpu.sync_copy(zeros_hbm.at[pl.ds(0, rows_pt)],
                        acc.at[pl.ds(rbase, rows_pt)])

        @pl.when(s == 0)
        def _():
            pltpu.sync_copy(zeros_hbm.at[pl.ds(0, tail)],
                            acc.at[pl.ds(NS * rows_pt, tail)])

        plsc.subcore_barrier()

        def start_gather(j, b):
            off = ebase + j * K
            pltpu.sync_copy(src_hbm.at[pl.ds(off, K)], srcv.at[b])
            pltpu.sync_copy(dst_hbm.at[pl.ds(off, K)], dstv.at[b])
            pltpu.async_copy(vals_hbm.at[srcv.at[b]], rowsv.at[b], sems.at[b])

        start_gather(0, 0)

        def body(j, carry):
            b = lax.rem(j, 2)
            nb = lax.rem(j + 1, 2)

            @pl.when(j + 1 < chunks)
            def _():
                start_gather(j + 1, nb)

            pltpu.make_async_copy(vals_hbm.at[srcv.at[b]], rowsv.at[b],
                                  sems.at[b]).wait()
            pltpu.sync_copy(rowsv.at[b], acc.at[dstv.at[b]], add=True)
            return carry

        lax.fori_loop(0, chunks, body, 0)
        plsc.subcore_barrier()
        pltpu.sync_copy(acc.at[pl.ds(rbase, rows_pt)],
                        out_hbm.at[c, pl.ds(rbase, rows_pt)])

        @pl.when(s == 0)
        def _():
            pltpu.sync_copy(acc.at[pl.ds(NS * rows_pt, tail)],
                            out_hbm.at[c, pl.ds(NS * rows_pt, tail)])

    return agg_kernel


def _dinv(degp_blk):
    deg = 1.0 + jnp.sum(degp_blk, axis=1)
    return lax.rsqrt(deg)[:, None]


def _tc1_body(x_ref, w1_ref, dp_ref, hs_ref):
    d = _dinv(dp_ref[...])
    h = jnp.dot(x_ref[...], w1_ref[...], preferred_element_type=jnp.float32)
    hs_ref[...] = h * d


def _tc2_body(a0_ref, a1_ref, hs_ref, dp_ref, b1_ref, out_ref):
    d = _dinv(dp_ref[...])
    z = jnp.maximum(d * (a0_ref[...] + a1_ref[...] + hs_ref[...]) + b1_ref[...],
                    0.0)
    out_ref[...] = z * d


def _tc3_body(a0_ref, a1_ref, zs_ref, dp_ref, w2_ref, b2_ref, out_ref):
    d = _dinv(dp_ref[...])
    g = d * (a0_ref[...] + a1_ref[...] + zs_ref[...])
    out_ref[...] = (jnp.dot(g, w2_ref[...], preferred_element_type=jnp.float32)
                    + b2_ref[...])


def kernel(x, edge_index, W1, b1, W2, b2):
    n, dx = x.shape
    e = edge_index.shape[1]
    h = W1.shape[1]
    c_out = W2.shape[1]
    src = edge_index[0]
    dst = edge_index[1]

    zeros_n = jnp.zeros((n,), jnp.float32)
    zeros_h = jnp.zeros((n // NS + 8, h), jnp.float32)
    b1r = b1.reshape(1, h)
    b2r = b2.reshape(1, c_out)

    bm = 1000
    grid = (n // bm,)
    row_blk = lambda w: pl.BlockSpec((bm, w), lambda i: (i, 0))
    dp_blk = pl.BlockSpec((bm, NW), lambda i: (i, 0))
    full_blk = lambda r, w: pl.BlockSpec((r, w), lambda i: (0, 0))

    deg_p = _make_deg_kernel(n, e)(dst, zeros_n).reshape(NW, n).T

    hs1 = pl.pallas_call(
        _tc1_body,
        grid=grid,
        in_specs=[row_blk(dx), full_blk(dx, h), dp_blk],
        out_specs=row_blk(h),
        out_shape=jax.ShapeDtypeStruct((n, h), jnp.float32),
    )(x, W1, deg_p)

    agg = _make_agg_kernel(n, e, h)
    acc1 = agg(hs1, src, dst, zeros_h)

    z1s = pl.pallas_call(
        _tc2_body,
        grid=grid,
        in_specs=[row_blk(h), row_blk(h), row_blk(h), dp_blk, full_blk(1, h)],
        out_specs=row_blk(h),
        out_shape=jax.ShapeDtypeStruct((n, h), jnp.float32),
    )(acc1[0], acc1[1], hs1, deg_p, b1r)

    acc2 = agg(z1s, src, dst, zeros_h)

    out = pl.pallas_call(
        _tc3_body,
        grid=grid,
        in_specs=[row_blk(h), row_blk(h), row_blk(h), dp_blk,
                  full_blk(h, c_out), full_blk(1, c_out)],
        out_specs=row_blk(c_out),
        out_shape=jax.ShapeDtypeStruct((n, c_out), jnp.float32),
    )(acc2[0], acc2[1], z1s, deg_p, W2, b2r)

    return out


# speedup vs baseline: 20.8987x; 1.0026x over previous
import functools

import jax
import jax.numpy as jnp
from jax import lax
from jax.experimental import pallas as pl
from jax.experimental.pallas import tpu as pltpu
from jax.experimental.pallas import tpu_sc as plsc

NC = 2
NS = 16
NW = NC * NS
K = 80
NB = 3
L = 16


def _make_deg_kernel(n_nodes, n_edges):
    ept = n_edges // NW

    mesh = plsc.VectorSubcoreMesh(core_axis_name="c", subcore_axis_name="s")

    @functools.partial(
        pl.kernel,
        out_type=jax.ShapeDtypeStruct((NW * n_nodes,), jnp.float32),
        mesh=mesh,
        scratch_types=[
            pltpu.VMEM((ept,), jnp.int32),
            pltpu.VMEM((n_nodes,), jnp.float32),
        ],
        compiler_params=pltpu.CompilerParams(needs_layout_passes=False),
    )
    def deg_kernel(dst_hbm, zeros_hbm, out_hbm, dstv, degl):
        c = lax.axis_index("c")
        s = lax.axis_index("s")
        w = c * NS + s
        pltpu.sync_copy(zeros_hbm, degl)
        pltpu.sync_copy(dst_hbm.at[pl.ds(w * ept, ept)], dstv)
        ones = jnp.full((L,), 1.0, jnp.float32)

        def body(i, carry):
            idx = dstv[pl.ds(i * L, L)]
            plsc.addupdate_scatter(degl, [idx], ones)
            return carry

        lax.fori_loop(0, ept // L, body, 0)
        pltpu.sync_copy(degl, out_hbm.at[pl.ds(w * n_nodes, n_nodes)])

    return deg_kernel


def _make_agg_kernel(n_nodes, n_edges, width):
    ept = n_edges // NW
    chunks = ept // K
    rows_pt = (n_nodes // NS) // 8 * 8
    tail = n_nodes - NS * rows_pt

    mesh = plsc.VectorSubcoreMesh(core_axis_name="c", subcore_axis_name="s")

    @functools.partial(
        pl.kernel,
        out_type=jax.ShapeDtypeStruct((NC, n_nodes, width), jnp.float32),
        mesh=mesh,
        scratch_types=[
            pltpu.VMEM((NB, K), jnp.int32),
            pltpu.VMEM((NB, K), jnp.int32),
            pltpu.VMEM((NB, K, width), jnp.float32),
            pltpu.VMEM_SHARED((n_nodes, width), jnp.float32),
            pltpu.SemaphoreType.DMA((NB,)),
        ],
    )
    def agg_kernel(vals_hbm, src_hbm, dst_hbm, zeros_hbm, out_hbm,
                   srcv, dstv, rowsv, acc, sems):
        c = lax.axis_index("c")
        s = lax.axis_index("s")
        ebase = (c * NS + s) * ept
        rbase = s * rows_pt
        pltpu.sync_copy(zeros_hbm.at[pl.ds(0, rows_pt)],
                        acc.at[pl.ds(rbase, rows_pt)])

        @pl.when(s == 0)
        def _():
            pltpu.sync_copy(zeros_hbm.at[pl.ds(0, tail)],
                            acc.at[pl.ds(NS * rows_pt, tail)])

        plsc.subcore_barrier()

        def load_and_gather(t, b):
            off = ebase + t * K
            pltpu.sync_copy(src_hbm.at[pl.ds(off, K)], srcv.at[b])
            pltpu.sync_copy(dst_hbm.at[pl.ds(off, K)], dstv.at[b])
            pltpu.async_copy(vals_hbm.at[srcv.at[b]], rowsv.at[b], sems.at[b])

        for t in range(NB):
            load_and_gather(t, t)

        def body(j, carry):
            b = lax.rem(j, NB)
            pltpu.make_async_copy(vals_hbm.at[srcv.at[b]], rowsv.at[b],
                                  sems.at[b]).wait()
            pltpu.sync_copy(rowsv.at[b], acc.at[dstv.at[b]], add=True)
            nxt = j + NB

            @pl.when(nxt < chunks)
            def _():
                load_and_gather(nxt, b)

            return carry

        lax.fori_loop(0, chunks, body, 0)
        plsc.subcore_barrier()
        pltpu.sync_copy(acc.at[pl.ds(rbase, rows_pt)],
                        out_hbm.at[c, pl.ds(rbase, rows_pt)])

        @pl.when(s == 0)
        def _():
            pltpu.sync_copy(acc.at[pl.ds(NS * rows_pt, tail)],
                            out_hbm.at[c, pl.ds(NS * rows_pt, tail)])

    return agg_kernel


def _dinv(degp_blk):
    deg = 1.0 + jnp.sum(degp_blk, axis=1)
    return lax.rsqrt(deg)[:, None]


def _tc1_body(x_ref, w1_ref, dp_ref, hs_ref):
    d = _dinv(dp_ref[...])
    h = jnp.dot(x_ref[...], w1_ref[...], preferred_element_type=jnp.float32)
    hs_ref[...] = h * d


def _tc2_body(a0_ref, a1_ref, hs_ref, dp_ref, b1_ref, out_ref):
    d = _dinv(dp_ref[...])
    z = jnp.maximum(d * (a0_ref[...] + a1_ref[...] + hs_ref[...]) + b1_ref[...],
                    0.0)
    out_ref[...] = z * d


def _tc3_body(a0_ref, a1_ref, zs_ref, dp_ref, w2_ref, b2_ref, out_ref):
    d = _dinv(dp_ref[...])
    g = d * (a0_ref[...] + a1_ref[...] + zs_ref[...])
    out_ref[...] = (jnp.dot(g, w2_ref[...], preferred_element_type=jnp.float32)
                    + b2_ref[...])


def kernel(x, edge_index, W1, b1, W2, b2):
    n, dx = x.shape
    e = edge_index.shape[1]
    h = W1.shape[1]
    c_out = W2.shape[1]
    src = edge_index[0]
    dst = edge_index[1]

    zeros_n = jnp.zeros((n,), jnp.float32)
    zeros_h = jnp.zeros((n // NS + 8, h), jnp.float32)
    b1r = b1.reshape(1, h)
    b2r = b2.reshape(1, c_out)

    bm = 1000
    grid = (n // bm,)
    row_blk = lambda w: pl.BlockSpec((bm, w), lambda i: (i, 0))
    dp_blk = pl.BlockSpec((bm, NW), lambda i: (i, 0))
    full_blk = lambda r, w: pl.BlockSpec((r, w), lambda i: (0, 0))

    deg_p = _make_deg_kernel(n, e)(dst, zeros_n).reshape(NW, n).T

    hs1 = pl.pallas_call(
        _tc1_body,
        grid=grid,
        in_specs=[row_blk(dx), full_blk(dx, h), dp_blk],
        out_specs=row_blk(h),
        out_shape=jax.ShapeDtypeStruct((n, h), jnp.float32),
    )(x, W1, deg_p)

    agg = _make_agg_kernel(n, e, h)
    acc1 = agg(hs1, src, dst, zeros_h)

    z1s = pl.pallas_call(
        _tc2_body,
        grid=grid,
        in_specs=[row_blk(h), row_blk(h), row_blk(h), dp_blk, full_blk(1, h)],
        out_specs=row_blk(h),
        out_shape=jax.ShapeDtypeStruct((n, h), jnp.float32),
    )(acc1[0], acc1[1], hs1, deg_p, b1r)

    acc2 = agg(z1s, src, dst, zeros_h)

    out = pl.pallas_call(
        _tc3_body,
        grid=grid,
        in_specs=[row_blk(h), row_blk(h), row_blk(h), dp_blk,
                  full_blk(h, c_out), full_blk(1, c_out)],
        out_specs=row_blk(c_out),
        out_shape=jax.ShapeDtypeStruct((n, c_out), jnp.float32),
    )(acc2[0], acc2[1], z1s, deg_p, W2, b2r)

    return out


# speedup vs baseline: 32.0220x; 1.5322x over previous
import functools

import jax
import jax.numpy as jnp
from jax import lax
from jax.experimental import pallas as pl
from jax.experimental.pallas import tpu as pltpu
from jax.experimental.pallas import tpu_sc as plsc

NC = 2
NS = 16
NW = NC * NS
K = 80
NB = 3
L = 16


def _make_deg_kernel(n_nodes, n_edges):
    ept = n_edges // NW

    mesh = plsc.VectorSubcoreMesh(core_axis_name="c", subcore_axis_name="s")

    @functools.partial(
        pl.kernel,
        out_type=jax.ShapeDtypeStruct((NW * n_nodes,), jnp.float32),
        mesh=mesh,
        scratch_types=[
            pltpu.VMEM((ept,), jnp.int32),
            pltpu.VMEM((n_nodes,), jnp.float32),
        ],
        compiler_params=pltpu.CompilerParams(needs_layout_passes=False),
    )
    def deg_kernel(dst_hbm, zeros_hbm, out_hbm, dstv, degl):
        c = lax.axis_index("c")
        s = lax.axis_index("s")
        w = c * NS + s
        pltpu.sync_copy(zeros_hbm, degl)
        pltpu.sync_copy(dst_hbm.at[pl.ds(w * ept, ept)], dstv)
        ones = jnp.full((L,), 1.0, jnp.float32)

        def body(i, carry):
            idx = dstv[pl.ds(i * L, L)]
            plsc.addupdate_scatter(degl, [idx], ones)
            return carry

        lax.fori_loop(0, ept // L, body, 0)
        pltpu.sync_copy(degl, out_hbm.at[pl.ds(w * n_nodes, n_nodes)])

    return deg_kernel


def _make_agg_kernel(n_nodes, n_edges, width):
    ept = n_edges // NW
    chunks = ept // K
    rows_pt = (n_nodes // NS) // 8 * 8
    tail = n_nodes - NS * rows_pt

    mesh = plsc.VectorSubcoreMesh(core_axis_name="c", subcore_axis_name="s")

    @functools.partial(
        pl.kernel,
        out_type=jax.ShapeDtypeStruct((NC, n_nodes, width), jnp.float32),
        mesh=mesh,
        scratch_types=[
            pltpu.VMEM((NB, K), jnp.int32),
            pltpu.VMEM((NB, K), jnp.int32),
            pltpu.VMEM((NB, K, width), jnp.float32),
            pltpu.VMEM_SHARED((n_nodes, width), jnp.float32),
            pltpu.SemaphoreType.DMA((NB,)),
        ],
    )
    def agg_kernel(vals_hbm, src_hbm, dst_hbm, zeros_hbm, out_hbm,
                   srcv, dstv, rowsv, acc, sems):
        c = lax.axis_index("c")
        s = lax.axis_index("s")
        ebase = (c * NS + s) * ept
        rbase = s * rows_pt
        pltpu.sync_copy(zeros_hbm.at[pl.ds(0, rows_pt)],
                        acc.at[pl.ds(rbase, rows_pt)])

        @pl.when(s == 0)
        def _():
            pltpu.sync_copy(zeros_hbm.at[pl.ds(0, tail)],
                            acc.at[pl.ds(NS * rows_pt, tail)])

        plsc.subcore_barrier()

        def load_and_gather(t, b):
            off = ebase + t * K
            pltpu.sync_copy(src_hbm.at[pl.ds(off, K)], srcv.at[b])
            pltpu.sync_copy(dst_hbm.at[pl.ds(off, K)], dstv.at[b])
            pltpu.async_copy(vals_hbm.at[srcv.at[b]], rowsv.at[b], sems.at[b])

        for t in range(NB):
            load_and_gather(t, t)

        def body(j, carry):
            b = lax.rem(j, NB)
            pltpu.make_async_copy(vals_hbm.at[srcv.at[b]], rowsv.at[b],
                                  sems.at[b]).wait()
            pltpu.sync_copy(rowsv.at[b], acc.at[dstv.at[b]], add=True)
            nxt = j + NB

            @pl.when(nxt < chunks)
            def _():
                load_and_gather(nxt, b)

            return carry

        lax.fori_loop(0, chunks, body, 0)
        plsc.subcore_barrier()
        pltpu.sync_copy(acc.at[pl.ds(rbase, rows_pt)],
                        out_hbm.at[c, pl.ds(rbase, rows_pt)])

        @pl.when(s == 0)
        def _():
            pltpu.sync_copy(acc.at[pl.ds(NS * rows_pt, tail)],
                            out_hbm.at[c, pl.ds(NS * rows_pt, tail)])

    return agg_kernel


def _make_vec2_kernel(n_nodes, n_pad, n_edges):
    ept = n_edges // NW

    mesh = plsc.VectorSubcoreMesh(core_axis_name="c", subcore_axis_name="s")

    @functools.partial(
        pl.kernel,
        out_type=jax.ShapeDtypeStruct((NW, 2, n_pad), jnp.float32),
        mesh=mesh,
        scratch_types=[
            pltpu.VMEM((ept,), jnp.int32),
            pltpu.VMEM((ept,), jnp.int32),
            pltpu.VMEM((2, n_pad), jnp.float32),
            pltpu.VMEM((2, n_pad), jnp.float32),
        ],
        compiler_params=pltpu.CompilerParams(needs_layout_passes=False),
    )
    def vec2_kernel(qt_hbm, src_hbm, dst_hbm, zeros2_hbm, out_hbm,
                    srcv, dstv, tab, accl):
        c = lax.axis_index("c")
        s = lax.axis_index("s")
        w = c * NS + s
        pltpu.sync_copy(zeros2_hbm, accl)
        pltpu.sync_copy(qt_hbm, tab)
        pltpu.sync_copy(src_hbm.at[pl.ds(w * ept, ept)], srcv)
        pltpu.sync_copy(dst_hbm.at[pl.ds(w * ept, ept)], dstv)
        zero16 = jnp.zeros((L,), jnp.int32)
        one16 = jnp.full((L,), 1, jnp.int32)

        def body(i, carry):
            src16 = srcv[pl.ds(i * L, L)]
            dst16 = dstv[pl.ds(i * L, L)]
            v0 = plsc.load_gather(tab, [zero16, src16])
            plsc.addupdate_scatter(accl, [zero16, dst16], v0)
            v1 = plsc.load_gather(tab, [one16, src16])
            plsc.addupdate_scatter(accl, [one16, dst16], v1)
            return carry

        lax.fori_loop(0, ept // L, body, 0)
        pltpu.sync_copy(accl, out_hbm.at[w])

    return vec2_kernel


def _dinv(degp_blk):
    deg = 1.0 + jnp.sum(degp_blk, axis=1)
    return lax.rsqrt(deg)[:, None]


def _tc1_body(x_ref, w1_ref, dp_ref, hs_ref):
    d = _dinv(dp_ref[...])
    h = jnp.dot(x_ref[...], w1_ref[...], preferred_element_type=jnp.float32)
    hs_ref[...] = h * d


def _tc2_body(a0_ref, a1_ref, hs_ref, dp_ref, w2_ref, b1_ref, out_ref):
    d = _dinv(dp_ref[...])
    z = jnp.maximum(d * (a0_ref[...] + a1_ref[...] + hs_ref[...]) + b1_ref[...],
                    0.0)
    qt = lax.dot_general(w2_ref[...], z, (((0,), (1,)), ((), ())),
                         preferred_element_type=jnp.float32)
    out_ref[...] = qt * d.reshape(1, -1)


def _tc3_body(ap_ref, qt_ref, dp_ref, b2_ref, out_ref):
    d = _dinv(dp_ref[...])
    agg2 = jnp.sum(ap_ref[...].reshape(NW, 2, -1), axis=0)
    out_ref[...] = d.reshape(1, -1) * (agg2 + qt_ref[...]) + b2_ref[...]


def kernel(x, edge_index, W1, b1, W2, b2):
    n, dx = x.shape
    e = edge_index.shape[1]
    h = W1.shape[1]
    c_out = W2.shape[1]
    src = edge_index[0]
    dst = edge_index[1]

    npad = (n + 1023) // 1024 * 1024
    zeros_n = jnp.zeros((n,), jnp.float32)
    zeros_h = jnp.zeros((n // NS + 8, h), jnp.float32)
    zeros_2p = jnp.zeros((2, npad), jnp.float32)
    b1r = b1.reshape(1, h)
    b2c = b2.reshape(c_out, 1)

    bm = 1024
    grid = (npad // bm,)
    row_blk = lambda w: pl.BlockSpec((bm, w), lambda i: (i, 0))
    col_blk = pl.BlockSpec((2, bm), lambda i: (0, i))
    dp_blk = pl.BlockSpec((bm, NW), lambda i: (i, 0))
    full_blk = lambda r, w: pl.BlockSpec((r, w), lambda i: (0, 0))

    deg_p = _make_deg_kernel(n, e)(dst, zeros_n).reshape(NW, n).T

    hs1 = pl.pallas_call(
        _tc1_body,
        grid=grid,
        in_specs=[row_blk(dx), full_blk(dx, h), dp_blk],
        out_specs=row_blk(h),
        out_shape=jax.ShapeDtypeStruct((n, h), jnp.float32),
    )(x, W1, deg_p)

    agg = _make_agg_kernel(n, e, h)
    acc1 = agg(hs1, src, dst, zeros_h)

    qt = pl.pallas_call(
        _tc2_body,
        grid=grid,
        in_specs=[row_blk(h), row_blk(h), row_blk(h), dp_blk,
                  full_blk(h, c_out), full_blk(1, h)],
        out_specs=col_blk,
        out_shape=jax.ShapeDtypeStruct((2, npad), jnp.float32),
    )(acc1[0], acc1[1], hs1, deg_p, W2, b1r)

    acc2 = _make_vec2_kernel(n, npad, e)(qt, src, dst, zeros_2p)
    acc2f = acc2.reshape(NW * 2, npad)

    out_t = pl.pallas_call(
        _tc3_body,
        grid=grid,
        in_specs=[pl.BlockSpec((NW * 2, bm), lambda i: (0, i)), col_blk,
                  dp_blk, full_blk(c_out, 1)],
        out_specs=col_blk,
        out_shape=jax.ShapeDtypeStruct((2, npad), jnp.float32),
    )(acc2f, qt, deg_p, b2c)

    return out_t[:, :n].T


# speedup vs baseline: 32.5275x; 1.0158x over previous
import functools

import jax
import jax.numpy as jnp
from jax import lax
from jax.experimental import pallas as pl
from jax.experimental.pallas import tpu as pltpu
from jax.experimental.pallas import tpu_sc as plsc

NC = 2
NS = 16
NW = NC * NS
K = 80
NB = 3
L = 16


def _make_deg_kernel(n_nodes, n_edges):
    ept = n_edges // NW

    mesh = plsc.VectorSubcoreMesh(core_axis_name="c", subcore_axis_name="s")

    @functools.partial(
        pl.kernel,
        out_type=jax.ShapeDtypeStruct((NW * n_nodes,), jnp.float32),
        mesh=mesh,
        scratch_types=[
            pltpu.VMEM((ept,), jnp.int32),
            pltpu.VMEM((n_nodes,), jnp.float32),
        ],
        compiler_params=pltpu.CompilerParams(needs_layout_passes=False),
    )
    def deg_kernel(dst_hbm, zeros_hbm, out_hbm, dstv, degl):
        c = lax.axis_index("c")
        s = lax.axis_index("s")
        w = c * NS + s
        pltpu.sync_copy(zeros_hbm, degl)
        pltpu.sync_copy(dst_hbm.at[pl.ds(w * ept, ept)], dstv)
        ones = jnp.full((L,), 1.0, jnp.float32)

        def body(i, carry):
            idx = dstv[pl.ds(i * L, L)]
            plsc.addupdate_scatter(degl, [idx], ones)
            return carry

        lax.fori_loop(0, ept // L, body, 0)
        pltpu.sync_copy(degl, out_hbm.at[pl.ds(w * n_nodes, n_nodes)])

    return deg_kernel


def _make_agg_kernel(n_nodes, n_edges, width):
    ept = n_edges // NW
    chunks = ept // K
    rows_pt = (n_nodes // NS) // 8 * 8
    tail = n_nodes - NS * rows_pt

    mesh = plsc.VectorSubcoreMesh(core_axis_name="c", subcore_axis_name="s")

    @functools.partial(
        pl.kernel,
        out_type=jax.ShapeDtypeStruct((NC, n_nodes, width), jnp.float32),
        mesh=mesh,
        scratch_types=[
            pltpu.VMEM((NB, K), jnp.int32),
            pltpu.VMEM((NB, K), jnp.int32),
            pltpu.VMEM((NB, K, width), jnp.float32),
            pltpu.VMEM_SHARED((n_nodes, width), jnp.float32),
            pltpu.SemaphoreType.DMA((NB,)),
        ],
    )
    def agg_kernel(vals_hbm, src_hbm, dst_hbm, zeros_hbm, out_hbm,
                   srcv, dstv, rowsv, acc, sems):
        c = lax.axis_index("c")
        s = lax.axis_index("s")
        ebase = (c * NS + s) * ept
        rbase = s * rows_pt
        pltpu.sync_copy(zeros_hbm.at[pl.ds(0, rows_pt)],
                        acc.at[pl.ds(rbase, rows_pt)])

        @pl.when(s == 0)
        def _():
            pltpu.sync_copy(zeros_hbm.at[pl.ds(0, tail)],
                            acc.at[pl.ds(NS * rows_pt, tail)])

        plsc.subcore_barrier()

        def load_and_gather(t, b):
            off = ebase + t * K
            pltpu.sync_copy(src_hbm.at[pl.ds(off, K)], srcv.at[b])
            pltpu.sync_copy(dst_hbm.at[pl.ds(off, K)], dstv.at[b])
            pltpu.async_copy(vals_hbm.at[srcv.at[b]], rowsv.at[b], sems.at[b])

        for t in range(NB):
            load_and_gather(t, t)

        def body(j, carry):
            b = lax.rem(j, NB)
            pltpu.make_async_copy(vals_hbm.at[srcv.at[b]], rowsv.at[b],
                                  sems.at[b]).wait()
            pltpu.sync_copy(rowsv.at[b], acc.at[dstv.at[b]], add=True)
            nxt = j + NB

            @pl.when(nxt < chunks)
            def _():
                load_and_gather(nxt, b)

            return carry

        lax.fori_loop(0, chunks, body, 0)
        plsc.subcore_barrier()
        pltpu.sync_copy(acc.at[pl.ds(rbase, rows_pt)],
                        out_hbm.at[c, pl.ds(rbase, rows_pt)])

        @pl.when(s == 0)
        def _():
            pltpu.sync_copy(acc.at[pl.ds(NS * rows_pt, tail)],
                            out_hbm.at[c, pl.ds(NS * rows_pt, tail)])

    return agg_kernel


def _make_vec2_kernel(n_nodes, n_pad, n_edges):
    ept = n_edges // NW

    mesh = plsc.VectorSubcoreMesh(core_axis_name="c", subcore_axis_name="s")

    @functools.partial(
        pl.kernel,
        out_type=jax.ShapeDtypeStruct((NW, 2, n_pad), jnp.float32),
        mesh=mesh,
        scratch_types=[
            pltpu.VMEM((ept,), jnp.int32),
            pltpu.VMEM((ept,), jnp.int32),
            pltpu.VMEM((2, n_pad), jnp.float32),
            pltpu.VMEM((2, n_pad), jnp.float32),
        ],
        compiler_params=pltpu.CompilerParams(needs_layout_passes=False),
    )
    def vec2_kernel(qt_hbm, src_hbm, dst_hbm, zeros2_hbm, out_hbm,
                    srcv, dstv, tab, accl):
        c = lax.axis_index("c")
        s = lax.axis_index("s")
        w = c * NS + s
        pltpu.sync_copy(zeros2_hbm, accl)
        pltpu.sync_copy(qt_hbm, tab)
        pltpu.sync_copy(src_hbm.at[pl.ds(w * ept, ept)], srcv)
        pltpu.sync_copy(dst_hbm.at[pl.ds(w * ept, ept)], dstv)
        zero16 = jnp.zeros((L,), jnp.int32)
        one16 = jnp.full((L,), 1, jnp.int32)

        def body(i, carry):
            src16 = srcv[pl.ds(i * L, L)]
            dst16 = dstv[pl.ds(i * L, L)]
            v0 = plsc.load_gather(tab, [zero16, src16])
            plsc.addupdate_scatter(accl, [zero16, dst16], v0)
            v1 = plsc.load_gather(tab, [one16, src16])
            plsc.addupdate_scatter(accl, [one16, dst16], v1)
            return carry

        lax.fori_loop(0, ept // L, body, 0)
        pltpu.sync_copy(accl, out_hbm.at[w])

    return vec2_kernel


def _dinv(degp_blk):
    deg = 1.0 + jnp.sum(degp_blk, axis=0)
    return lax.rsqrt(deg)[:, None]


def _tc1_body(x_ref, w1_ref, dp_ref, hs_ref):
    d = _dinv(dp_ref[...])
    h = jnp.dot(x_ref[...], w1_ref[...], preferred_element_type=jnp.float32)
    hs_ref[...] = h * d


def _tc2_body(a0_ref, a1_ref, hs_ref, dp_ref, w2_ref, b1_ref, out_ref):
    d = _dinv(dp_ref[...])
    z = jnp.maximum(d * (a0_ref[...] + a1_ref[...] + hs_ref[...]) + b1_ref[...],
                    0.0)
    qt = lax.dot_general(w2_ref[...], z, (((0,), (1,)), ((), ())),
                         preferred_element_type=jnp.float32)
    out_ref[...] = qt * d.reshape(1, -1)


def _tc3_body(ap_ref, qt_ref, dp_ref, b2_ref, out_ref):
    d = _dinv(dp_ref[...])
    agg2 = jnp.sum(ap_ref[...].reshape(NW, 2, -1), axis=0)
    out_t = d.reshape(1, -1) * (agg2 + qt_ref[...]) + b2_ref[...]
    out_ref[...] = out_t.T


def kernel(x, edge_index, W1, b1, W2, b2):
    n, dx = x.shape
    e = edge_index.shape[1]
    h = W1.shape[1]
    c_out = W2.shape[1]
    src = edge_index[0]
    dst = edge_index[1]

    npad = (n + 1023) // 1024 * 1024
    zeros_n = jnp.zeros((n,), jnp.float32)
    zeros_h = jnp.zeros((n // NS + 8, h), jnp.float32)
    zeros_2p = jnp.zeros((2, npad), jnp.float32)
    b1r = b1.reshape(1, h)
    b2c = b2.reshape(c_out, 1)

    bm = 1024
    grid = (npad // bm,)
    row_blk = lambda w: pl.BlockSpec((bm, w), lambda i: (i, 0))
    col_blk = pl.BlockSpec((2, bm), lambda i: (0, i))
    dp_blk = pl.BlockSpec((NW, bm), lambda i: (0, i))
    full_blk = lambda r, w: pl.BlockSpec((r, w), lambda i: (0, 0))

    deg_p = _make_deg_kernel(n, e)(dst, zeros_n).reshape(NW, n)

    hs1 = pl.pallas_call(
        _tc1_body,
        grid=grid,
        in_specs=[row_blk(dx), full_blk(dx, h), dp_blk],
        out_specs=row_blk(h),
        out_shape=jax.ShapeDtypeStruct((n, h), jnp.float32),
    )(x, W1, deg_p)

    agg = _make_agg_kernel(n, e, h)
    acc1 = agg(hs1, src, dst, zeros_h)

    qt = pl.pallas_call(
        _tc2_body,
        grid=grid,
        in_specs=[row_blk(h), row_blk(h), row_blk(h), dp_blk,
                  full_blk(h, c_out), full_blk(1, h)],
        out_specs=col_blk,
        out_shape=jax.ShapeDtypeStruct((2, npad), jnp.float32),
    )(acc1[0], acc1[1], hs1, deg_p, W2, b1r)

    acc2 = _make_vec2_kernel(n, npad, e)(qt, src, dst, zeros_2p)
    acc2f = acc2.reshape(NW * 2, npad)

    out_t = pl.pallas_call(
        _tc3_body,
        grid=grid,
        in_specs=[pl.BlockSpec((NW * 2, bm), lambda i: (0, i)), col_blk,
                  dp_blk, full_blk(c_out, 1)],
        out_specs=pl.BlockSpec((bm, c_out), lambda i: (i, 0)),
        out_shape=jax.ShapeDtypeStruct((n, c_out), jnp.float32),
    )(acc2f, qt, deg_p, b2c)

    return out_t


# speedup vs baseline: 36.5724x; 1.1244x over previous
import functools

import jax
import jax.numpy as jnp
from jax import lax
from jax.experimental import pallas as pl
from jax.experimental.pallas import tpu as pltpu
from jax.experimental.pallas import tpu_sc as plsc

NC = 2
NS = 16
NW = NC * NS
K = 128
NB = 2
L = 16


def _make_deg_kernel(n_nodes, n_edges):
    ept = n_edges // NW

    mesh = plsc.VectorSubcoreMesh(core_axis_name="c", subcore_axis_name="s")

    @functools.partial(
        pl.kernel,
        out_type=jax.ShapeDtypeStruct((NW * n_nodes,), jnp.float32),
        mesh=mesh,
        scratch_types=[
            pltpu.VMEM((ept,), jnp.int32),
            pltpu.VMEM((n_nodes,), jnp.float32),
        ],
        compiler_params=pltpu.CompilerParams(needs_layout_passes=False),
    )
    def deg_kernel(dst_hbm, zeros_hbm, out_hbm, dstv, degl):
        c = lax.axis_index("c")
        s = lax.axis_index("s")
        w = c * NS + s
        pltpu.sync_copy(zeros_hbm, degl)
        pltpu.sync_copy(dst_hbm.at[pl.ds(w * ept, ept)], dstv)
        ones = jnp.full((L,), 1.0, jnp.float32)

        def body(i, carry):
            idx = dstv[pl.ds(i * L, L)]
            plsc.addupdate_scatter(degl, [idx], ones)
            return carry

        lax.fori_loop(0, ept // L, body, 0)
        pltpu.sync_copy(degl, out_hbm.at[pl.ds(w * n_nodes, n_nodes)])

    return deg_kernel


def _make_agg_kernel(n_nodes, n_edges, width):
    ept = n_edges // NW
    chunks = ept // K
    kt = ept - chunks * K
    rows_pt = (n_nodes // NS) // 8 * 8
    tail = n_nodes - NS * rows_pt

    mesh = plsc.VectorSubcoreMesh(core_axis_name="c", subcore_axis_name="s")

    @functools.partial(
        pl.kernel,
        out_type=jax.ShapeDtypeStruct((NC, n_nodes, width), jnp.float32),
        mesh=mesh,
        scratch_types=[
            pltpu.VMEM((NB, K), jnp.int32),
            pltpu.VMEM((NB, K), jnp.int32),
            pltpu.VMEM((NB, K, width), jnp.float32),
            pltpu.VMEM((max(kt, 8),), jnp.int32),
            pltpu.VMEM((max(kt, 8),), jnp.int32),
            pltpu.VMEM_SHARED((n_nodes, width), jnp.float32),
            pltpu.SemaphoreType.DMA((NB,)),
        ],
    )
    def agg_kernel(vals_hbm, src_hbm, dst_hbm, zeros_hbm, out_hbm,
                   srcv, dstv, rowsv, srct, dstt, acc, sems):
        c = lax.axis_index("c")
        s = lax.axis_index("s")
        ebase = (c * NS + s) * ept
        rbase = s * rows_pt
        pltpu.sync_copy(zeros_hbm.at[pl.ds(0, rows_pt)],
                        acc.at[pl.ds(rbase, rows_pt)])

        @pl.when(s == 0)
        def _():
            pltpu.sync_copy(zeros_hbm.at[pl.ds(0, tail)],
                            acc.at[pl.ds(NS * rows_pt, tail)])

        plsc.subcore_barrier()

        def load_and_gather(t, b):
            off = ebase + t * K
            pltpu.sync_copy(src_hbm.at[pl.ds(off, K)], srcv.at[b])
            pltpu.sync_copy(dst_hbm.at[pl.ds(off, K)], dstv.at[b])
            pltpu.async_copy(vals_hbm.at[srcv.at[b]], rowsv.at[b], sems.at[b])

        for t in range(NB):
            load_and_gather(t, t)

        def body(j, carry):
            b = lax.rem(j, NB)
            pltpu.make_async_copy(vals_hbm.at[srcv.at[b]], rowsv.at[b],
                                  sems.at[b]).wait()
            pltpu.sync_copy(rowsv.at[b], acc.at[dstv.at[b]], add=True)
            nxt = j + NB

            @pl.when(nxt < chunks)
            def _():
                load_and_gather(nxt, b)

            return carry

        lax.fori_loop(0, chunks, body, 0)

        if kt:
            toff = ebase + chunks * K
            pltpu.sync_copy(src_hbm.at[pl.ds(toff, kt)], srct)
            pltpu.sync_copy(dst_hbm.at[pl.ds(toff, kt)], dstt)
            pltpu.async_copy(vals_hbm.at[srct], rowsv.at[0, pl.ds(0, kt)],
                             sems.at[0]).wait()
            pltpu.sync_copy(rowsv.at[0, pl.ds(0, kt)], acc.at[dstt], add=True)

        plsc.subcore_barrier()
        pltpu.sync_copy(acc.at[pl.ds(rbase, rows_pt)],
                        out_hbm.at[c, pl.ds(rbase, rows_pt)])

        @pl.when(s == 0)
        def _():
            pltpu.sync_copy(acc.at[pl.ds(NS * rows_pt, tail)],
                            out_hbm.at[c, pl.ds(NS * rows_pt, tail)])

    return agg_kernel


def _make_vec2_kernel(n_nodes, n_pad, n_edges):
    ept = n_edges // NW

    mesh = plsc.VectorSubcoreMesh(core_axis_name="c", subcore_axis_name="s")

    @functools.partial(
        pl.kernel,
        out_type=jax.ShapeDtypeStruct((NW, 2, n_pad), jnp.float32),
        mesh=mesh,
        scratch_types=[
            pltpu.VMEM((ept,), jnp.int32),
            pltpu.VMEM((ept,), jnp.int32),
            pltpu.VMEM((2, n_pad), jnp.float32),
            pltpu.VMEM((2, n_pad), jnp.float32),
        ],
        compiler_params=pltpu.CompilerParams(needs_layout_passes=False),
    )
    def vec2_kernel(qt_hbm, src_hbm, dst_hbm, zeros2_hbm, out_hbm,
                    srcv, dstv, tab, accl):
        c = lax.axis_index("c")
        s = lax.axis_index("s")
        w = c * NS + s
        pltpu.sync_copy(zeros2_hbm, accl)
        pltpu.sync_copy(qt_hbm, tab)
        pltpu.sync_copy(src_hbm.at[pl.ds(w * ept, ept)], srcv)
        pltpu.sync_copy(dst_hbm.at[pl.ds(w * ept, ept)], dstv)
        zero16 = jnp.zeros((L,), jnp.int32)
        one16 = jnp.full((L,), 1, jnp.int32)

        def body(i, carry):
            src16 = srcv[pl.ds(i * L, L)]
            dst16 = dstv[pl.ds(i * L, L)]
            v0 = plsc.load_gather(tab, [zero16, src16])
            plsc.addupdate_scatter(accl, [zero16, dst16], v0)
            v1 = plsc.load_gather(tab, [one16, src16])
            plsc.addupdate_scatter(accl, [one16, dst16], v1)
            return carry

        lax.fori_loop(0, ept // L, body, 0)
        pltpu.sync_copy(accl, out_hbm.at[w])

    return vec2_kernel


def _dinv(degp_blk):
    deg = 1.0 + jnp.sum(degp_blk, axis=0)
    return lax.rsqrt(deg)[:, None]


def _tc1_body(x_ref, w1_ref, dp_ref, hs_ref):
    d = _dinv(dp_ref[...])
    h = jnp.dot(x_ref[...], w1_ref[...], preferred_element_type=jnp.float32)
    hs_ref[...] = h * d


def _tc2_body(a0_ref, a1_ref, hs_ref, dp_ref, w2_ref, b1_ref, out_ref):
    d = _dinv(dp_ref[...])
    z = jnp.maximum(d * (a0_ref[...] + a1_ref[...] + hs_ref[...]) + b1_ref[...],
                    0.0)
    qt = lax.dot_general(w2_ref[...], z, (((0,), (1,)), ((), ())),
                         preferred_element_type=jnp.float32)
    out_ref[...] = qt * d.reshape(1, -1)


def _tc3_body(ap_ref, qt_ref, dp_ref, b2_ref, out_ref):
    d = _dinv(dp_ref[...])
    agg2 = jnp.sum(ap_ref[...].reshape(NW, 2, -1), axis=0)
    out_t = d.reshape(1, -1) * (agg2 + qt_ref[...]) + b2_ref[...]
    out_ref[...] = out_t.T


def kernel(x, edge_index, W1, b1, W2, b2):
    n, dx = x.shape
    e = edge_index.shape[1]
    h = W1.shape[1]
    c_out = W2.shape[1]
    src = edge_index[0]
    dst = edge_index[1]

    npad = (n + 1023) // 1024 * 1024
    zeros_n = jnp.zeros((n,), jnp.float32)
    zeros_h = jnp.zeros((n // NS + 8, h), jnp.float32)
    zeros_2p = jnp.zeros((2, npad), jnp.float32)
    b1r = b1.reshape(1, h)
    b2c = b2.reshape(c_out, 1)

    bm = 1024
    grid = (npad // bm,)
    row_blk = lambda w: pl.BlockSpec((bm, w), lambda i: (i, 0))
    col_blk = pl.BlockSpec((2, bm), lambda i: (0, i))
    dp_blk = pl.BlockSpec((NW, bm), lambda i: (0, i))
    full_blk = lambda r, w: pl.BlockSpec((r, w), lambda i: (0, 0))

    deg_p = _make_deg_kernel(n, e)(dst, zeros_n).reshape(NW, n)

    hs1 = pl.pallas_call(
        _tc1_body,
        grid=grid,
        in_specs=[row_blk(dx), full_blk(dx, h), dp_blk],
        out_specs=row_blk(h),
        out_shape=jax.ShapeDtypeStruct((n, h), jnp.float32),
    )(x, W1, deg_p)

    agg = _make_agg_kernel(n, e, h)
    acc1 = agg(hs1, src, dst, zeros_h)

    qt = pl.pallas_call(
        _tc2_body,
        grid=grid,
        in_specs=[row_blk(h), row_blk(h), row_blk(h), dp_blk,
                  full_blk(h, c_out), full_blk(1, h)],
        out_specs=col_blk,
        out_shape=jax.ShapeDtypeStruct((2, npad), jnp.float32),
    )(acc1[0], acc1[1], hs1, deg_p, W2, b1r)

    acc2 = _make_vec2_kernel(n, npad, e)(qt, src, dst, zeros_2p)
    acc2f = acc2.reshape(NW * 2, npad)

    out_t = pl.pallas_call(
        _tc3_body,
        grid=grid,
        in_specs=[pl.BlockSpec((NW * 2, bm), lambda i: (0, i)), col_blk,
                  dp_blk, full_blk(c_out, 1)],
        out_specs=pl.BlockSpec((bm, c_out), lambda i: (i, 0)),
        out_shape=jax.ShapeDtypeStruct((n, c_out), jnp.float32),
    )(acc2f, qt, deg_p, b2c)

    return out_t
